# baseline (speedup 1.0000x reference)
import functools

import jax
import jax.numpy as jnp
from jax import lax
from jax.experimental import pallas as pl
from jax.experimental.pallas import tpu as pltpu
from jax.experimental.pallas import tpu_sc as plsc


def _gather_kernel(n_per_w, n_super, K, D, NC,
                   idx_hbm, table_hbm, out_hbm, idx_v, rows_v, sem):
    wid = lax.axis_index("s") * NC + lax.axis_index("c")
    sc_rows = K * 128
    base128 = wid * (n_per_w // 128)

    def body(c, carry):
        r0 = base128 + c * K
        pltpu.sync_copy(idx_hbm.at[pl.ds(r0, K)], idx_v)
        copies = []
        for j in range(K):
            copies.append(
                pltpu.async_copy(
                    table_hbm.at[idx_v.at[j]],
                    rows_v.at[pl.ds(j * 128, 128)],
                    sem,
                )
            )
        for cp in copies:
            cp.wait()
        pltpu.sync_copy(rows_v, out_hbm.at[pl.ds(r0 * 128, sc_rows)])
        return carry

    lax.fori_loop(0, n_super, body, 0)


def kernel(x, table):
    B, L = x.shape
    V, D = table.shape
    N = B * L
    idx = x.reshape(N // 128, 128).astype(jnp.int32)

    info = plsc.get_sparse_core_info()
    NC, NS = info.num_cores, info.num_subcores
    NW = NC * NS
    assert N % (NW * 128) == 0
    n_per_w = N // NW

    K = 8
    sc_rows = K * 128
    assert n_per_w % sc_rows == 0
    n_super = n_per_w // sc_rows

    mesh = plsc.VectorSubcoreMesh(core_axis_name="c", subcore_axis_name="s")
    grid_kernel = pl.kernel(
        functools.partial(_gather_kernel, n_per_w, n_super, K, D, NC),
        mesh=mesh,
        out_type=jax.ShapeDtypeStruct((N, D), jnp.float32),
        scratch_types=[
            pltpu.VMEM((K, 128), jnp.int32),
            pltpu.VMEM((sc_rows, D), jnp.float32),
            pltpu.SemaphoreType.DMA,
        ],
        compiler_params=pltpu.CompilerParams(use_tc_tiling_on_sc=False),
    )
    out = grid_kernel(idx, table)
    return out.reshape(B, L, D)


# speedup vs baseline: 1.0176x; 1.0176x over previous
import functools

import jax
import jax.numpy as jnp
from jax import lax
from jax.experimental import pallas as pl
from jax.experimental.pallas import tpu as pltpu
from jax.experimental.pallas import tpu_sc as plsc


def _gather_kernel(n_per_w, n_super, K, D, NC,
                   idx_hbm, table_hbm, out_hbm,
                   idx_v, rows_v, gsem0, gsem1, ssem0, ssem1):
    wid = lax.axis_index("s") * NC + lax.axis_index("c")
    sc_rows = K * 128
    base128 = wid * (n_per_w // 128)
    gsems = (gsem0, gsem1)
    ssems = (ssem0, ssem1)

    pltpu.sync_copy(idx_hbm.at[pl.ds(base128, n_per_w // 128)], idx_v)

    def fire_gathers(c, b):
        for j in range(K):
            pltpu.async_copy(
                table_hbm.at[idx_v.at[c * K + j]],
                rows_v.at[b].at[pl.ds(j * 128, 128)],
                gsems[b],
            )

    def drain_gathers(c, b):
        for j in range(K):
            pltpu.make_async_copy(
                table_hbm.at[idx_v.at[c * K + j]],
                rows_v.at[b].at[pl.ds(j * 128, 128)],
                gsems[b],
            ).wait()

    def fire_scatter(c, b):
        pltpu.async_copy(
            rows_v.at[b],
            out_hbm.at[pl.ds((base128 + c * K) * 128, sc_rows)],
            ssems[b],
        )

    def drain_scatter(c, b):
        pltpu.make_async_copy(
            rows_v.at[b],
            out_hbm.at[pl.ds((base128 + c * K) * 128, sc_rows)],
            ssems[b],
        ).wait()

    fire_gathers(0, 0)
    fire_gathers(1, 1)
    drain_gathers(0, 0)
    fire_scatter(0, 0)
    drain_scatter(0, 0)
    fire_gathers(2, 0)
    drain_gathers(1, 1)
    fire_scatter(1, 1)

    def pair(g, carry):
        c = 2 * g
        drain_scatter(c - 1, 1)
        fire_gathers(c + 1, 1)
        drain_gathers(c, 0)
        fire_scatter(c, 0)

        drain_scatter(c, 0)
        fire_gathers(c + 2, 0)
        drain_gathers(c + 1, 1)
        fire_scatter(c + 1, 1)
        return carry

    lax.fori_loop(1, n_super // 2 - 1, pair, 0)

    c = n_super - 2
    drain_scatter(c - 1, 1)
    fire_gathers(c + 1, 1)
    drain_gathers(c, 0)
    fire_scatter(c, 0)
    drain_scatter(c, 0)
    drain_gathers(c + 1, 1)
    fire_scatter(c + 1, 1)
    drain_scatter(c + 1, 1)


def kernel(x, table):
    B, L = x.shape
    V, D = table.shape
    N = B * L
    idx = x.reshape(N // 128, 128).astype(jnp.int32)

    info = plsc.get_sparse_core_info()
    NC, NS = info.num_cores, info.num_subcores
    NW = NC * NS
    assert N % (NW * 128) == 0
    n_per_w = N // NW

    K = 5
    sc_rows = K * 128
    assert n_per_w % sc_rows == 0
    n_super = n_per_w // sc_rows
    assert n_super % 2 == 0 and n_super >= 4

    mesh = plsc.VectorSubcoreMesh(core_axis_name="c", subcore_axis_name="s")
    grid_kernel = pl.kernel(
        functools.partial(_gather_kernel, n_per_w, n_super, K, D, NC),
        mesh=mesh,
        out_type=jax.ShapeDtypeStruct((N, D), jnp.float32),
        scratch_types=[
            pltpu.VMEM((n_per_w // 128, 128), jnp.int32),
            pltpu.VMEM((2, sc_rows, D), jnp.float32),
            pltpu.SemaphoreType.DMA,
            pltpu.SemaphoreType.DMA,
            pltpu.SemaphoreType.DMA,
            pltpu.SemaphoreType.DMA,
        ],
        compiler_params=pltpu.CompilerParams(use_tc_tiling_on_sc=False),
    )
    out = grid_kernel(idx, table)
    return out.reshape(B, L, D)
